# Initial kernel scaffold; baseline (speedup 1.0000x reference)
import jax
import jax.numpy as jnp
from jax.experimental import pallas as pl


def kernel(x, bag_sizes, W1, b1, W2, b2):
    raise NotImplementedError("write your pallas kernel here")


# speedup vs baseline: 6.3096x; 6.3096x over previous
import functools

import jax
import jax.numpy as jnp
from jax.experimental import pallas as pl
from jax.experimental.pallas import tpu as pltpu

_TOKENS = 32768
_D_IN = 1024
_D_H = 512
_BAGS = 16
_BLK = 512
_NBLK = _TOKENS // _BLK

_NEG_INF = float("-inf")


def _attn_body(ends_sref, starts_ref, ends_ref, x_ref, w1_ref, b1_ref,
               w2_ref, out_ref, m_ref, d_ref, acc_ref):
    i = pl.program_id(0)
    total = ends_sref[_BAGS - 1]

    @pl.when(i == 0)
    def _init():
        m_ref[...] = jnp.full(m_ref.shape, _NEG_INF, jnp.float32)
        d_ref[...] = jnp.zeros(d_ref.shape, jnp.float32)
        acc_ref[...] = jnp.zeros(acc_ref.shape, jnp.float32)

    @pl.when(i * _BLK < total)
    def _compute():
        xb = x_ref[...]
        h = jnp.tanh(
            jnp.dot(xb, w1_ref[...], preferred_element_type=jnp.float32)
            + b1_ref[...])
        s = jnp.sum(h * w2_ref[...], axis=1, keepdims=True)

        tok = i * _BLK + jax.lax.broadcasted_iota(jnp.int32, (_BLK, _BAGS), 0)
        mask = (tok >= starts_ref[...]) & (tok < ends_ref[...])
        sb = jnp.where(mask, s, _NEG_INF)
        mb = jnp.max(sb, axis=0, keepdims=True)
        m_old = m_ref[...]
        m_new = jnp.maximum(m_old, mb)
        scale = jnp.where(m_new == _NEG_INF, 0.0, jnp.exp(m_old - m_new))
        e = jnp.where(mask, jnp.exp(sb - m_new), 0.0)
        m_ref[...] = m_new
        d_ref[...] = d_ref[...] * scale + jnp.sum(e, axis=0, keepdims=True)
        acc_ref[...] = acc_ref[...] * scale + jax.lax.dot_general(
            xb, e, (((0,), (0,)), ((), ())),
            preferred_element_type=jnp.float32)

    @pl.when(i == _NBLK - 1)
    def _finish():
        d = d_ref[...]
        out_ref[...] = jnp.where(d > 0.0, acc_ref[...] / d, 0.0)


def _x_map(i, ends):
    total = ends[_BAGS - 1]
    last = jnp.maximum((total + _BLK - 1) // _BLK - 1, 0)
    return (jnp.minimum(i, last), 0)


@jax.jit
def _attn_pool(x, starts2d, ends2d, ends, w1, b1r, w2r):
    grid_spec = pltpu.PrefetchScalarGridSpec(
        num_scalar_prefetch=1,
        grid=(_NBLK,),
        in_specs=[
            pl.BlockSpec((1, _BAGS), lambda i, e: (0, 0)),
            pl.BlockSpec((1, _BAGS), lambda i, e: (0, 0)),
            pl.BlockSpec((_BLK, _D_IN), _x_map),
            pl.BlockSpec((_D_IN, _D_H), lambda i, e: (0, 0)),
            pl.BlockSpec((1, _D_H), lambda i, e: (0, 0)),
            pl.BlockSpec((1, _D_H), lambda i, e: (0, 0)),
        ],
        out_specs=pl.BlockSpec((_D_IN, _BAGS), lambda i, e: (0, 0)),
        scratch_shapes=[
            pltpu.VMEM((1, _BAGS), jnp.float32),
            pltpu.VMEM((1, _BAGS), jnp.float32),
            pltpu.VMEM((_D_IN, _BAGS), jnp.float32),
        ],
    )
    return pl.pallas_call(
        _attn_body,
        grid_spec=grid_spec,
        out_shape=jax.ShapeDtypeStruct((_D_IN, _BAGS), jnp.float32),
        compiler_params=pltpu.CompilerParams(
            dimension_semantics=("arbitrary",)),
    )(ends, starts2d, ends2d, x, w1, b1r, w2r)


def kernel(x, bag_sizes, W1, b1, W2, b2):
    ends = jnp.cumsum(bag_sizes, dtype=jnp.int32)
    starts = ends - bag_sizes
    out_t = _attn_pool(
        x,
        starts.reshape(1, _BAGS),
        ends.reshape(1, _BAGS),
        ends,
        W1,
        b1.reshape(1, _D_H),
        W2.reshape(1, _D_H),
    )
    return out_t.T
